# baseline (speedup 1.0000x reference)
import functools

import jax
import jax.numpy as jnp
from jax.experimental import pallas as pl
from jax.experimental.pallas import tpu as pltpu
from jax.experimental.pallas import tpu_sc as plsc

B = 16384
D_IMG = 2048
D_TXT = 768
D_USR = 32

NW = 32
SPLIT_IMG = 4
SPLIT_TXT = 3
CI = 128

BLK = 512


def _sc_gather(item4, item3, user_id, img_tab, txt_tab, usr_tab):
    mesh = plsc.VectorSubcoreMesh(core_axis_name="c", subcore_axis_name="s")
    n4 = B * SPLIT_IMG // NW
    n3 = B * SPLIT_TXT // NW
    nu = B // NW
    d4 = D_IMG // SPLIT_IMG
    d3 = D_TXT // SPLIT_TXT
    du = usr_tab.shape[1]
    out_type = (
        jax.ShapeDtypeStruct((B * SPLIT_IMG, d4), jnp.float32),
        jax.ShapeDtypeStruct((B * SPLIT_TXT, d3), jnp.float32),
        jax.ShapeDtypeStruct((B, du), jnp.float32),
    )

    @functools.partial(
        pl.kernel, out_type=out_type, mesh=mesh,
        scratch_types=[
            pltpu.VMEM((n4 // CI, CI), jnp.int32),
            pltpu.VMEM((n3 // CI, CI), jnp.int32),
            pltpu.VMEM((nu // CI, CI), jnp.int32),
            pltpu.VMEM((CI, d4), jnp.float32),
            pltpu.VMEM((CI, d3), jnp.float32),
            pltpu.VMEM((CI, du), jnp.float32),
        ])
    def gather_kernel(item4_hbm, item3_hbm, user_hbm, img_hbm, txt_hbm,
                      usr_hbm, img_out, txt_out, usr_out,
                      idx4_v, idx3_v, idxu_v, img_buf, txt_buf, usr_buf):
        wid = jax.lax.axis_index("c") * 16 + jax.lax.axis_index("s")
        pltpu.sync_copy(item4_hbm.at[wid], idx4_v)
        pltpu.sync_copy(item3_hbm.at[wid], idx3_v)
        pltpu.sync_copy(user_hbm.at[wid], idxu_v)

        def run(nchunks, idx_v, tab, buf, out_hbm, rows_per_worker):
            base = wid * rows_per_worker

            @pl.loop(0, nchunks)
            def _(c):
                pltpu.sync_copy(tab.at[idx_v.at[c]], buf)
                pltpu.sync_copy(buf, out_hbm.at[pl.ds(base + c * CI, CI)])

        run(n4 // CI, idx4_v, img_hbm, img_buf, img_out, n4)
        run(n3 // CI, idx3_v, txt_hbm, txt_buf, txt_out, n3)
        run(nu // CI, idxu_v, usr_hbm, usr_buf, usr_out, nu)

    return gather_kernel(
        item4.reshape(NW, n4 // CI, CI),
        item3.reshape(NW, n3 // CI, CI),
        user_id.reshape(NW, nu // CI, CI),
        img_tab, txt_tab, usr_tab)


def _mlp_body(img_ref, txt_ref, usr_ref, w1a_ref, w1b_ref, b1_ref,
              w2_ref, b2_ref, w3u_ref, w3i_ref, b3_ref, w4_ref, b4_ref,
              w5_ref, b5_ref, out_ref):
    f32 = jnp.float32
    bf16 = jnp.bfloat16
    h = jnp.dot(img_ref[...].astype(bf16), w1a_ref[...],
                preferred_element_type=f32)
    h = h + jnp.dot(txt_ref[...].astype(bf16), w1b_ref[...],
                    preferred_element_type=f32)
    h = jax.nn.relu(h + b1_ref[...])
    ie = jax.nn.relu(
        jnp.dot(h.astype(bf16), w2_ref[...], preferred_element_type=f32)
        + b2_ref[...])
    h2 = jnp.dot(usr_ref[:, :D_USR].astype(bf16), w3u_ref[...],
                 preferred_element_type=f32)
    h2 = h2 + jnp.dot(ie.astype(bf16), w3i_ref[...],
                      preferred_element_type=f32)
    h2 = jax.nn.relu(h2 + b3_ref[...])
    h3 = jax.nn.relu(
        jnp.dot(h2.astype(bf16), w4_ref[...], preferred_element_type=f32)
        + b4_ref[...])
    logit = jnp.dot(h3.astype(bf16), w5_ref[...],
                    preferred_element_type=f32) + b5_ref[...]
    out_ref[...] = jax.nn.sigmoid(logit)


def _tc_mlp(img_g, txt_g, usr_g, w1, b1, w2, b2, w3, b3, w4, b4, w5, b5):
    bf16 = jnp.bfloat16
    w1a = w1[:D_IMG].astype(bf16)
    w1b = w1[D_IMG:].astype(bf16)
    w3u = w3[:D_USR].astype(bf16)
    w3i = w3[D_USR:].astype(bf16)
    weights = [w1a, w1b, b1.reshape(1, -1), w2.astype(bf16),
               b2.reshape(1, -1), w3u, w3i, b3.reshape(1, -1),
               w4.astype(bf16), b4.reshape(1, -1), w5.astype(bf16),
               b5.reshape(1, -1)]

    batch_spec = lambda d: pl.BlockSpec((BLK, d), lambda i: (i, 0))
    const_spec = lambda a: pl.BlockSpec(a.shape, lambda i: (0, 0))
    out = pl.pallas_call(
        _mlp_body,
        grid=(B // BLK,),
        in_specs=[batch_spec(D_IMG), batch_spec(D_TXT),
                  batch_spec(usr_g.shape[1])]
                 + [const_spec(a) for a in weights],
        out_specs=pl.BlockSpec((BLK, 1), lambda i: (i, 0)),
        out_shape=jax.ShapeDtypeStruct((B, 1), jnp.float32),
    )(img_g, txt_g, usr_g, *weights)
    return out.reshape(-1)


def kernel(user_id, item_id, rating, user_table, text_table, image_table,
           w1, b1, w2, b2, w3, b3, w4, b4, w5, b5):
    item4 = (item_id[:, None] * SPLIT_IMG
             + jnp.arange(SPLIT_IMG, dtype=item_id.dtype)).reshape(-1)
    item3 = (item_id[:, None] * SPLIT_TXT
             + jnp.arange(SPLIT_TXT, dtype=item_id.dtype)).reshape(-1)
    img_tab = image_table.reshape(-1, D_IMG // SPLIT_IMG)
    txt_tab = text_table.reshape(-1, D_TXT // SPLIT_TXT)
    usr_tab = jnp.pad(user_table, ((0, 0), (0, 128 - D_USR)))
    img_o, txt_o, usr_g = _sc_gather(item4, item3, user_id, img_tab,
                                     txt_tab, usr_tab)
    img_g = img_o.reshape(B, D_IMG)
    txt_g = txt_o.reshape(B, D_TXT)
    pred = _tc_mlp(img_g, txt_g, usr_g, w1, b1, w2, b2, w3, b3, w4, b4,
                   w5, b5)
    return (user_id, pred, rating.astype(jnp.float32))


# speedup vs baseline: 1.1503x; 1.1503x over previous
import functools

import jax
import jax.numpy as jnp
from jax.experimental import pallas as pl
from jax.experimental.pallas import tpu as pltpu
from jax.experimental.pallas import tpu_sc as plsc

B = 16384
D_IMG = 2048
D_TXT = 768
D_USR = 32

NW = 32
SPLIT_IMG = 4
SPLIT_TXT = 3
CI = 128

BLK = 512


def _sc_gather(item4, item3, user_id, img_tab, txt_tab, usr_tab):
    mesh = plsc.VectorSubcoreMesh(core_axis_name="c", subcore_axis_name="s")
    n4 = B * SPLIT_IMG // NW
    n3 = B * SPLIT_TXT // NW
    nu = B // NW
    d4 = D_IMG // SPLIT_IMG
    d3 = D_TXT // SPLIT_TXT
    du = usr_tab.shape[1]
    out_type = (
        jax.ShapeDtypeStruct((B * SPLIT_IMG, d4), jnp.float32),
        jax.ShapeDtypeStruct((B * SPLIT_TXT, d3), jnp.float32),
        jax.ShapeDtypeStruct((B, du), jnp.float32),
    )

    @functools.partial(
        pl.kernel, out_type=out_type, mesh=mesh,
        scratch_types=[
            pltpu.VMEM((n4 // CI, CI), jnp.int32),
            pltpu.VMEM((n3 // CI, CI), jnp.int32),
            pltpu.VMEM((nu // CI, CI), jnp.int32),
            pltpu.VMEM((CI, d4), jnp.float32),
            pltpu.VMEM((CI, d3), jnp.float32),
            pltpu.VMEM((CI, du), jnp.float32),
        ])
    def gather_kernel(item4_hbm, item3_hbm, user_hbm, img_hbm, txt_hbm,
                      usr_hbm, img_out, txt_out, usr_out,
                      idx4_v, idx3_v, idxu_v, img_buf, txt_buf, usr_buf):
        wid = jax.lax.axis_index("c") * 16 + jax.lax.axis_index("s")
        pltpu.sync_copy(item4_hbm.at[wid], idx4_v)
        pltpu.sync_copy(item3_hbm.at[wid], idx3_v)
        pltpu.sync_copy(user_hbm.at[wid], idxu_v)

        def run(nchunks, idx_v, tab, buf, out_hbm, rows_per_worker):
            base = wid * rows_per_worker

            @pl.loop(0, nchunks)
            def _(c):
                pltpu.sync_copy(tab.at[idx_v.at[c]], buf)
                pltpu.sync_copy(buf, out_hbm.at[pl.ds(base + c * CI, CI)])

        run(n4 // CI, idx4_v, img_hbm, img_buf, img_out, n4)
        run(n3 // CI, idx3_v, txt_hbm, txt_buf, txt_out, n3)
        run(nu // CI, idxu_v, usr_hbm, usr_buf, usr_out, nu)

    return gather_kernel(
        item4.reshape(NW, n4 // CI, CI),
        item3.reshape(NW, n3 // CI, CI),
        user_id.reshape(NW, nu // CI, CI),
        img_tab, txt_tab, usr_tab)


def _mlp_body(*refs):
    (img0, img1, img2, img3, txt0, txt1, txt2, usr_ref,
     w1a0, w1a1, w1a2, w1a3, w1b0, w1b1, w1b2, b1_ref,
     w2_ref, b2_ref, w3u_ref, w3i_ref, b3_ref, w4_ref, b4_ref,
     w5_ref, b5_ref, out_ref) = refs
    f32 = jnp.float32
    bf16 = jnp.bfloat16
    h = jnp.dot(img0[...].astype(bf16), w1a0[...],
                preferred_element_type=f32)
    for part, w in ((img1, w1a1), (img2, w1a2), (img3, w1a3),
                    (txt0, w1b0), (txt1, w1b1), (txt2, w1b2)):
        h = h + jnp.dot(part[...].astype(bf16), w[...],
                        preferred_element_type=f32)
    h = jax.nn.relu(h + b1_ref[...])
    ie = jax.nn.relu(
        jnp.dot(h.astype(bf16), w2_ref[...], preferred_element_type=f32)
        + b2_ref[...])
    h2 = jnp.dot(usr_ref[:, :D_USR].astype(bf16), w3u_ref[...],
                 preferred_element_type=f32)
    h2 = h2 + jnp.dot(ie.astype(bf16), w3i_ref[...],
                      preferred_element_type=f32)
    h2 = jax.nn.relu(h2 + b3_ref[...])
    h3 = jax.nn.relu(
        jnp.dot(h2.astype(bf16), w4_ref[...], preferred_element_type=f32)
        + b4_ref[...])
    logit = jnp.dot(h3.astype(bf16), w5_ref[...],
                    preferred_element_type=f32) + b5_ref[...]
    out_ref[...] = jax.nn.sigmoid(logit)


def _tc_mlp(img_o, txt_o, usr_g, w1, b1, w2, b2, w3, b3, w4, b4, w5, b5):
    bf16 = jnp.bfloat16
    d4 = D_IMG // SPLIT_IMG
    d3 = D_TXT // SPLIT_TXT
    w1a = [w1[d4 * p:d4 * (p + 1)].astype(bf16) for p in range(SPLIT_IMG)]
    w1b = [w1[D_IMG + d3 * q:D_IMG + d3 * (q + 1)].astype(bf16)
           for q in range(SPLIT_TXT)]
    w3u = w3[:D_USR].astype(bf16)
    w3i = w3[D_USR:].astype(bf16)
    weights = w1a + w1b + [b1.reshape(1, -1), w2.astype(bf16),
                           b2.reshape(1, -1), w3u, w3i, b3.reshape(1, -1),
                           w4.astype(bf16), b4.reshape(1, -1),
                           w5.astype(bf16), b5.reshape(1, -1)]

    nblk = B // BLK
    part_spec = lambda d, p: pl.BlockSpec(
        (BLK, d), lambda i, p=p: (p * nblk + i, 0))
    const_spec = lambda a: pl.BlockSpec(a.shape, lambda i: (0, 0))
    out = pl.pallas_call(
        _mlp_body,
        grid=(nblk,),
        in_specs=[part_spec(d4, p) for p in range(SPLIT_IMG)]
                 + [part_spec(d3, q) for q in range(SPLIT_TXT)]
                 + [pl.BlockSpec((BLK, usr_g.shape[1]), lambda i: (i, 0))]
                 + [const_spec(a) for a in weights],
        out_specs=pl.BlockSpec((BLK, 1), lambda i: (i, 0)),
        out_shape=jax.ShapeDtypeStruct((B, 1), jnp.float32),
    )(*([img_o] * SPLIT_IMG + [txt_o] * SPLIT_TXT + [usr_g] + weights))
    return out.reshape(-1)


def kernel(user_id, item_id, rating, user_table, text_table, image_table,
           w1, b1, w2, b2, w3, b3, w4, b4, w5, b5):
    item4 = (item_id[None, :] * SPLIT_IMG
             + jnp.arange(SPLIT_IMG, dtype=item_id.dtype)[:, None]
             ).reshape(-1)
    item3 = (item_id[None, :] * SPLIT_TXT
             + jnp.arange(SPLIT_TXT, dtype=item_id.dtype)[:, None]
             ).reshape(-1)
    img_tab = image_table.reshape(-1, D_IMG // SPLIT_IMG)
    txt_tab = text_table.reshape(-1, D_TXT // SPLIT_TXT)
    usr_tab = jnp.pad(user_table, ((0, 0), (0, 128 - D_USR)))
    img_o, txt_o, usr_g = _sc_gather(item4, item3, user_id, img_tab,
                                     txt_tab, usr_tab)
    pred = _tc_mlp(img_o, txt_o, usr_g, w1, b1, w2, b2, w3, b3, w4, b4,
                   w5, b5)
    return (user_id, pred, rating.astype(jnp.float32))
